# Initial kernel scaffold; baseline (speedup 1.0000x reference)
import jax
import jax.numpy as jnp
from jax.experimental import pallas as pl


def kernel(input, emb, W_ih, W_hh, b_ih, b_hh):
    raise NotImplementedError("write your pallas kernel here")


# speedup vs baseline: 4.2478x; 4.2478x over previous
import functools

import jax
import jax.numpy as jnp
from jax.experimental import pallas as pl
from jax.experimental.pallas import tpu as pltpu

NCELL = 4
VOCAB = 16
EMB = 512
HID = 512
BATCH = 64
SEQ = 64
G4 = 4 * HID


def _table_kernel(emb_ref, wih_ref, bih_ref, bhh_ref, table_ref):
    vrow = jax.lax.broadcasted_iota(jnp.int32, (VOCAB, 1), 0)
    acc = jnp.zeros((VOCAB, G4), dtype=jnp.float32)
    for j in range(NCELL):
        gj = jax.lax.dot_general(
            emb_ref[...], wih_ref[j],
            (((1,), (1,)), ((), ())),
            preferred_element_type=jnp.float32,
        ) + bih_ref[j][None, :] + bhh_ref[j][None, :]
        mask = (vrow % NCELL == j).astype(jnp.float32)
        acc = acc + mask * gj
    table_ref[...] = acc


def _lstm_kernel(tok_ref, table_ref, wstack_ref, out_ref, h_ref, c_ref):
    t = pl.program_id(0)

    @pl.when(t == 0)
    def _init():
        h_ref[...] = jnp.zeros((BATCH, HID), jnp.float32)
        c_ref[...] = jnp.zeros((BATCH, HID), jnp.float32)

    tok = tok_ref[0]
    vcol = jax.lax.broadcasted_iota(jnp.int32, (BATCH, VOCAB), 1)
    onehot = (tok == vcol).astype(jnp.float32)
    gx = jnp.dot(onehot, table_ref[...], preferred_element_type=jnp.float32)

    assign = tok % NCELL
    h = h_ref[...]
    hbig = jnp.concatenate(
        [h * (assign == j).astype(jnp.float32) for j in range(NCELL)], axis=1
    )
    gates = gx + jnp.dot(hbig, wstack_ref[...],
                         preferred_element_type=jnp.float32)

    i_g = jax.nn.sigmoid(gates[:, 0 * HID:1 * HID])
    f_g = jax.nn.sigmoid(gates[:, 1 * HID:2 * HID])
    g_g = jnp.tanh(gates[:, 2 * HID:3 * HID])
    o_g = jax.nn.sigmoid(gates[:, 3 * HID:4 * HID])
    c_new = f_g * c_ref[...] + i_g * g_g
    h_new = o_g * jnp.tanh(c_new)
    h_ref[...] = h_new
    c_ref[...] = c_new

    @pl.when(t == SEQ - 1)
    def _emit():
        out_ref[:, 0:HID] = h_new
        out_ref[:, HID:2 * HID] = c_new


@jax.jit
def kernel(input, emb, W_ih, W_hh, b_ih, b_hh):
    tokens = jnp.swapaxes(input.astype(jnp.int32), 0, 1).reshape(SEQ, BATCH, 1)

    table = pl.pallas_call(
        _table_kernel,
        out_shape=jax.ShapeDtypeStruct((VOCAB, G4), jnp.float32),
    )(emb, W_ih, b_ih, b_hh)

    wstack = jnp.transpose(W_hh, (0, 2, 1)).reshape(NCELL * HID, G4)

    out = pl.pallas_call(
        _lstm_kernel,
        grid=(SEQ,),
        in_specs=[
            pl.BlockSpec((1, BATCH, 1), lambda t: (t, 0, 0)),
            pl.BlockSpec((VOCAB, G4), lambda t: (0, 0)),
            pl.BlockSpec((NCELL * HID, G4), lambda t: (0, 0)),
        ],
        out_specs=pl.BlockSpec((BATCH, 2 * HID), lambda t: (0, 0)),
        out_shape=jax.ShapeDtypeStruct((BATCH, 2 * HID), jnp.float32),
        scratch_shapes=[
            pltpu.VMEM((BATCH, HID), jnp.float32),
            pltpu.VMEM((BATCH, HID), jnp.float32),
        ],
    )(tokens, table, wstack)
    return out
